# baseline (speedup 1.0000x reference)
import functools

import jax
import jax.numpy as jnp
from jax import lax
from jax.experimental import pallas as pl
from jax.experimental.pallas import tpu as pltpu
from jax.experimental.pallas import tpu_sc as plsc

N = 10000
E = 320000
D = 128
D_OUT = 40
DD = 128
NP = 10240
NHALF = NP // 2
ZR = NHALF
NC, NS = 2, 16
NW = NC * NS
K = 128
NBUF = 3
LAG = 2
NCH = 80
EPT = NCH * K
EP = NW * EPT
SEGC = 48
SEGK = SEGC * K
RPW = NHALF // NS
NB = 10
R = NP // NB

_MESH = plsc.VectorSubcoreMesh(core_axis_name="c", subcore_axis_name="s")


def _part_kernel(srcc, dstc, es, ed, cnts):
    def scoped(in_s, in_d, lo_s, lo_d, hi_s, hi_d, cnt_v):
        cid = lax.axis_index("c")
        sid = lax.axis_index("s")
        wid = sid * NC + cid
        pltpu.sync_copy(srcc.at[wid], in_s)
        pltpu.sync_copy(dstc.at[wid], in_d)
        fill_d = lax.iota(jnp.int32, 16) * 317

        def fill(i, carry):
            sl = pl.ds(i * 16, 16)
            lo_s[sl] = jnp.full((16,), N, jnp.int32)
            hi_s[sl] = jnp.full((16,), N, jnp.int32)
            lo_d[sl] = fill_d
            hi_d[sl] = fill_d
            return carry

        lax.fori_loop(0, EPT // 16, fill, 0)

        def body(i, carry):
            lo, hi = carry
            j = i // (K // 16)
            k = i % (K // 16)
            s16 = in_s[j, pl.ds(k * 16, 16)]
            d16 = in_d[j, pl.ds(k * 16, 16)]
            m = d16 < NHALF
            plsc.store_compressed(lo_s.at[pl.ds(lo, 16)], s16, mask=m)
            plsc.store_compressed(lo_d.at[pl.ds(lo, 16)], d16, mask=m)
            nm = jnp.logical_not(m)
            plsc.store_compressed(hi_s.at[pl.ds(hi, 16)], s16, mask=nm)
            plsc.store_compressed(hi_d.at[pl.ds(hi, 16)], d16 - NHALF,
                                  mask=nm)
            c = plsc.all_reduce_population_count(m)[0]
            return lo + c, hi + (16 - c)

        lo, hi = lax.fori_loop(0, EPT // 16, body, (0, 0))
        iota16 = lax.iota(jnp.int32, 16)
        cnt_v[...] = jnp.where(iota16 == 0, lo,
                               jnp.where(iota16 == 1, hi, 0))
        pltpu.sync_copy(cnt_v, cnts.at[pl.ds(wid * 16, 16)])
        pltpu.sync_copy(lo_s.at[pl.ds(0, SEGK)],
                        es.at[pl.ds(wid * SEGK, SEGK)])
        pltpu.sync_copy(lo_d.at[pl.ds(0, SEGK)],
                        ed.at[pl.ds(wid * SEGK, SEGK)])
        pltpu.sync_copy(hi_s.at[pl.ds(0, SEGK)],
                        es.at[pl.ds((NW + wid) * SEGK, SEGK)])
        pltpu.sync_copy(hi_d.at[pl.ds(0, SEGK)],
                        ed.at[pl.ds((NW + wid) * SEGK, SEGK)])

    pl.run_scoped(scoped,
                  pltpu.VMEM((NCH, K), jnp.int32),
                  pltpu.VMEM((NCH, K), jnp.int32),
                  *[pltpu.VMEM((EPT + 16,), jnp.int32) for _ in range(4)],
                  pltpu.VMEM((16,), jnp.int32))


def _make_part():
    return functools.partial(
        pl.kernel,
        out_type=(
            jax.ShapeDtypeStruct((2 * NW * SEGK,), jnp.int32),
            jax.ShapeDtypeStruct((2 * NW * SEGK,), jnp.int32),
            jax.ShapeDtypeStruct((NW * 16,), jnp.int32),
        ),
        mesh=_MESH,
        compiler_params=pltpu.CompilerParams(needs_layout_passes=False),
    )(_part_kernel)


def _spmm_kernel(width, es, ed, cnts, y, s_out,
                 sv0, dv0, sv1, dv1,
                 r0, r1, r2, g0, g1, g2, t0, t1, t2, z_sh):
    def scoped(cb0, cb1):
        _spmm_inner(width, es, ed, cnts, y, s_out,
                    ((sv0, dv0, cb0), (sv1, dv1, cb1)),
                    (r0, r1, r2), (g0, g1, g2), (t0, t1, t2),
                    z_sh)

    pl.run_scoped(scoped, pltpu.VMEM((16,), jnp.int32),
                  pltpu.VMEM((16,), jnp.int32))


def _spmm_inner(width, es, ed, cnts, y, s_out, segs, rows, gsem, ssem, z_sh):
    del width
    cid = lax.axis_index("c")
    sid = lax.axis_index("s")
    row0 = sid * RPW

    def gather(src_v, j, b):
        pltpu.async_copy(y.at[src_v.at[j]], rows[b], gsem[b])

    def gather_wait(b):
        pltpu.make_async_copy(y.at[pl.ds(0, K)], rows[b], gsem[b]).wait()

    def scatter(dst_v, j, b):
        pltpu.async_copy(rows[b], z_sh.at[dst_v.at[j]], ssem[b], add=True)

    def scatter_wait(b):
        pltpu.make_async_copy(rows[b], z_sh.at[pl.ds(0, K)], ssem[b]).wait()

    for q, (src_v, dst_v, smem) in enumerate(segs):
        w = 2 * sid + q
        pltpu.sync_copy(es.at[cid, w], src_v)
        pltpu.sync_copy(ed.at[cid, w], dst_v)
        pltpu.sync_copy(cnts.at[pl.ds(w * 16, 16)], smem)

    pltpu.sync_copy(y.at[pl.ds(cid * NHALF + row0, RPW)],
                    z_sh.at[pl.ds(row0, RPW)])
    plsc.subcore_barrier()

    for q, (src_v, dst_v, smem) in enumerate(segs):
        v = smem[...]
        cnt = jnp.where(cid == 0, v[0], v[1])
        nch = (cnt + K - 1) // K

        def body(g, carry, src_v=src_v, dst_v=dst_v, nch=nch):
            j0 = g * NBUF
            for off in range(NBUF):
                j = j0 + off
                bb = (off + NBUF - LAG) % NBUF

                @pl.when(jnp.logical_and(j >= NBUF, j < nch + NBUF))
                def _():
                    scatter_wait(off)

                @pl.when(j < nch)
                def _():
                    gather(src_v, j, off)

                jj = j - LAG

                @pl.when(jnp.logical_and(jj >= 0, jj < nch))
                def _():
                    gather_wait(bb)
                    scatter(dst_v, jj, bb)

            return carry

        nslot = (nch + 2 * NBUF - 1) // NBUF
        lax.fori_loop(0, nslot, body, 0)

    plsc.subcore_barrier()
    pltpu.sync_copy(z_sh.at[pl.ds(row0, RPW)],
                    s_out.at[pl.ds(cid * NHALF + row0, RPW)])


def _make_spmm(width):
    return functools.partial(
        pl.kernel,
        out_type=jax.ShapeDtypeStruct((NP, width), jnp.float32),
        mesh=_MESH,
        scratch_types=(
            [pltpu.VMEM((SEGC, K), jnp.int32) for _ in range(4)]
            + [pltpu.VMEM((K, width), jnp.float32) for _ in range(NBUF)]
            + [pltpu.SemaphoreType.DMA for _ in range(2 * NBUF)]
            + [pltpu.VMEM_SHARED((ZR, width), jnp.float32)]
        ),
        compiler_params=pltpu.CompilerParams(needs_layout_passes=False),
    )(functools.partial(_spmm_kernel, width))


def _row_mask(width):
    base = pl.program_id(0) * R
    rows = base + lax.broadcasted_iota(jnp.int32, (R, 1), 0)
    return rows < N


def _tc_pre_body(deg_ref, x_ref, w_ref, dinv_ref, y0_ref):
    dv = lax.rsqrt(jnp.maximum(deg_ref[:, 0:1], 1.0))
    dinv_ref[...] = jnp.broadcast_to(dv, (R, D))
    u = jnp.dot(x_ref[...], w_ref[...], preferred_element_type=jnp.float32)
    y0_ref[...] = jnp.where(_row_mask(D), u * dv, 0.0)


def _tc_pre(degs, x_p, w_enc):
    return pl.pallas_call(
        _tc_pre_body,
        grid=(NB,),
        in_specs=[
            pl.BlockSpec((R, D), lambda i: (i, 0)),
            pl.BlockSpec((R, D), lambda i: (i, 0)),
            pl.BlockSpec((D, D), lambda i: (0, 0)),
        ],
        out_specs=[
            pl.BlockSpec((R, D), lambda i: (i, 0)),
            pl.BlockSpec((R, D), lambda i: (i, 0)),
        ],
        out_shape=[
            jax.ShapeDtypeStruct((NP, D), jnp.float32),
            jax.ShapeDtypeStruct((NP, D), jnp.float32),
        ],
    )(degs, x_p, w_enc)


def _tc_stage(s, dinv, b, h_prev, w_next, *, smooth, use_relu, width,
              width_next):
    have_h = h_prev is not None
    have_w = w_next is not None

    def body(*refs):
        i = 0
        s_ref = refs[i]; i += 1
        dinv_ref = refs[i]; i += 1
        b_ref = refs[i]; i += 1
        h_ref = refs[i] if have_h else None
        i += have_h
        w_ref = refs[i] if have_w else None
        i += have_w
        out_refs = refs[i:]
        dv = dinv_ref[...]
        c = s_ref[...] * dv[:, :width] + b_ref[...]
        a = jnp.maximum(c, 0.0) if use_relu else c
        h = smooth * h_ref[...] + (1.0 - smooth) * a if have_h else a
        out_refs[0][...] = h
        if have_w:
            u = jnp.dot(h, w_ref[...], preferred_element_type=jnp.float32)
            out_refs[1][...] = jnp.where(_row_mask(width_next),
                                         u * dv[:, :width_next], 0.0)

    in_specs = [
        pl.BlockSpec((R, width), lambda i: (i, 0)),
        pl.BlockSpec((R, D), lambda i: (i, 0)),
        pl.BlockSpec((1, width), lambda i: (0, 0)),
    ]
    args = [s, dinv, b]
    if have_h:
        in_specs.append(pl.BlockSpec((R, width), lambda i: (i, 0)))
        args.append(h_prev)
    if have_w:
        in_specs.append(pl.BlockSpec((width, width_next), lambda i: (0, 0)))
        args.append(w_next)
    out_specs = [pl.BlockSpec((R, width), lambda i: (i, 0))]
    out_shape = [jax.ShapeDtypeStruct((NP, width), jnp.float32)]
    if have_w:
        out_specs.append(pl.BlockSpec((R, width_next), lambda i: (i, 0)))
        out_shape.append(jax.ShapeDtypeStruct((NP, width_next), jnp.float32))
    res = pl.pallas_call(
        body, grid=(NB,), in_specs=in_specs, out_specs=out_specs,
        out_shape=out_shape,
    )(*args)
    return res if have_w else (res[0], None)


def kernel(x, edge_index, W_enc, b_enc, W_gc, b_gc, W_dec, b_dec):
    schedule = (0.5, 0.5, 0.5, 0.5)
    src = edge_index[0].astype(jnp.int32)
    dst = edge_index[1].astype(jnp.int32)
    ppw = (EP - E) // NW
    pad = (N + jnp.arange(EP - E, dtype=jnp.int32) % (NP - N)).reshape(NW, ppw)
    srcc = jnp.concatenate([src.reshape(NW, E // NW), pad],
                           axis=1).reshape(NW, NCH, K)
    dstc = jnp.concatenate([dst.reshape(NW, E // NW), pad],
                           axis=1).reshape(NW, NCH, K)
    x_p = jnp.pad(x, ((0, NP - N), (0, 0)))
    w_dec_p = jnp.pad(W_dec, ((0, 0), (0, DD - D_OUT)))
    b_dec_p = jnp.pad(b_dec, (0, DD - D_OUT)).reshape(1, DD)
    b_enc2 = b_enc.reshape(1, D)
    b_gc2 = b_gc.reshape(1, D)
    ones_np = jnp.pad(jnp.ones((N, D), jnp.float32), ((0, NP - N), (0, 0)))

    es, ed, cnts = _make_part()(srcc, dstc)
    es = es.reshape(2, NW, SEGC, K)
    ed = ed.reshape(2, NW, SEGC, K)

    spmm = _make_spmm(D)

    degs = spmm(es, ed, cnts, ones_np)
    dinv, y = _tc_pre(degs, x_p, W_enc)

    s = spmm(es, ed, cnts, y)
    h, y = _tc_stage(s, dinv, b_enc2, None, W_gc,
                     smooth=0.0, use_relu=True, width=D, width_next=D)
    for it, sf in enumerate(schedule):
        last = it == len(schedule) - 1
        w_next = w_dec_p if last else W_gc
        wn = DD if last else D
        s = spmm(es, ed, cnts, y)
        h, y = _tc_stage(s, dinv, b_gc2, h, w_next,
                         smooth=sf, use_relu=True, width=D, width_next=wn)
    s = spmm(es, ed, cnts, y)
    out, _ = _tc_stage(s, dinv, b_dec_p, None, None,
                       smooth=0.0, use_relu=False, width=DD, width_next=DD)
    return out[:N, :D_OUT]


# speedup vs baseline: 2.0608x; 2.0608x over previous
import functools

import jax
import jax.numpy as jnp
from jax import lax
from jax.experimental import pallas as pl
from jax.experimental.pallas import tpu as pltpu
from jax.experimental.pallas import tpu_sc as plsc

N = 10000
E = 320000
D = 128
D_OUT = 40
DD = 128
NP = 10240
NC, NS = 2, 16
NW = NC * NS
K = 128
NBUF = 2
LAG = 1
NCH = 80
NH = 2
CH = NCH // NH
EP = NW * NCH * K
RPT = NP // NS
NB = 5
R = NP // NB

_MESH = plsc.VectorSubcoreMesh(core_axis_name="c", subcore_axis_name="s")


def _spmm_kernel(width, srcc, dstc, y, s_out, src_v, dst_v,
                 r0, r1, g0, g1, t0, t1, z_sh):
    del width
    rows = (r0, r1)
    gsem = (g0, g1)
    ssem = (t0, t1)
    cid = lax.axis_index("c")
    sid = lax.axis_index("s")
    wid = sid * NC + cid
    row0 = sid * RPT

    def gather(j, b):
        pltpu.async_copy(y.at[src_v.at[j]], rows[b], gsem[b])

    def gather_wait(b):
        pltpu.make_async_copy(y.at[pl.ds(0, K)], rows[b], gsem[b]).wait()

    def scatter(j, b):
        pltpu.async_copy(rows[b], z_sh.at[dst_v.at[j]], ssem[b], add=True)

    def scatter_wait(b):
        pltpu.make_async_copy(rows[b], z_sh.at[pl.ds(0, K)], ssem[b]).wait()

    pltpu.sync_copy(y.at[pl.ds(row0, RPT)], z_sh.at[pl.ds(row0, RPT)])
    plsc.subcore_barrier()

    for h in range(NH):
        pltpu.sync_copy(srcc.at[wid, pl.ds(h * CH, CH)], src_v)
        pltpu.sync_copy(dstc.at[wid, pl.ds(h * CH, CH)], dst_v)

        def body(g, carry):
            j0 = g * NBUF
            for off in range(NBUF):
                j = j0 + off
                bb = (off + NBUF - LAG) % NBUF

                @pl.when(jnp.logical_and(j >= NBUF, j < CH + NBUF))
                def _():
                    scatter_wait(off)

                @pl.when(j < CH)
                def _():
                    gather(j, off)

                jj = j - LAG

                @pl.when(jnp.logical_and(jj >= 0, jj < CH))
                def _():
                    gather_wait(bb)
                    scatter(jj, bb)

            return carry

        nslot = (CH + NBUF + NBUF - 1) // NBUF
        lax.fori_loop(0, nslot, body, 0)

    plsc.subcore_barrier()
    pltpu.sync_copy(z_sh.at[pl.ds(row0, RPT)],
                    s_out.at[cid, pl.ds(row0, RPT)])


def _make_spmm(width):
    return functools.partial(
        pl.kernel,
        out_type=jax.ShapeDtypeStruct((NC, NP, width), jnp.float32),
        mesh=_MESH,
        scratch_types=(
            [pltpu.VMEM((CH, K), jnp.int32), pltpu.VMEM((CH, K), jnp.int32)]
            + [pltpu.VMEM((K, width), jnp.float32) for _ in range(NBUF)]
            + [pltpu.SemaphoreType.DMA for _ in range(2 * NBUF)]
            + [pltpu.VMEM_SHARED((NP, width), jnp.float32)]
        ),
    )(functools.partial(_spmm_kernel, width))


def _deg_kernel(dstc, ones, deg_out, dst_v, r0, r1, g0, t0, t1, z_sh):
    rows = (r0, r1)
    ssem = (t0, t1)
    cid = lax.axis_index("c")
    sid = lax.axis_index("s")
    wid = sid * NC + cid
    row0 = sid * RPT

    def scatter(j, b):
        pltpu.async_copy(rows[b], z_sh.at[dst_v.at[j]], ssem[b], add=True)

    def scatter_wait(b):
        pltpu.make_async_copy(rows[b], z_sh.at[pl.ds(0, K)], ssem[b]).wait()

    pltpu.sync_copy(ones.at[pl.ds(0, K)], rows[0])
    pltpu.sync_copy(ones.at[pl.ds(0, K)], rows[1])
    pltpu.sync_copy(ones.at[pl.ds(row0, RPT)], z_sh.at[pl.ds(row0, RPT)])
    plsc.subcore_barrier()

    for h in range(NH):
        pltpu.sync_copy(dstc.at[wid, pl.ds(h * CH, CH)], dst_v)

        def body(g, carry):
            j0 = g * NBUF
            for off in range(NBUF):
                j = j0 + off

                @pl.when(jnp.logical_and(j >= NBUF, j < CH + NBUF))
                def _():
                    scatter_wait(off)

                @pl.when(j < CH)
                def _():
                    scatter(j, off)

            return carry

        lax.fori_loop(0, (CH + NBUF + NBUF - 1) // NBUF, body, 0)

    plsc.subcore_barrier()
    pltpu.sync_copy(z_sh.at[pl.ds(row0, RPT)],
                    deg_out.at[cid, pl.ds(row0, RPT)])


def _make_deg():
    return functools.partial(
        pl.kernel,
        out_type=jax.ShapeDtypeStruct((NC, NP, D), jnp.float32),
        mesh=_MESH,
        scratch_types=(
            [pltpu.VMEM((CH, K), jnp.int32)]
            + [pltpu.VMEM((K, D), jnp.float32) for _ in range(NBUF)]
            + [pltpu.SemaphoreType.DMA for _ in range(1 + NBUF)]
            + [pltpu.VMEM_SHARED((NP, D), jnp.float32)]
        ),
    )(_deg_kernel)


def _tc_pre_body(deg_ref, x_ref, w_ref, dinv_ref, y0_ref):
    dsum = deg_ref[0, :, 0:1] + deg_ref[1, :, 0:1]
    dv = lax.rsqrt(dsum - 1.0)
    dinv_ref[...] = jnp.broadcast_to(dv, (R, D))
    u = jnp.dot(x_ref[...], w_ref[...], preferred_element_type=jnp.float32)
    y0_ref[...] = u * dv


def _tc_pre(degs, x_p, w_enc):
    return pl.pallas_call(
        _tc_pre_body,
        grid=(NB,),
        in_specs=[
            pl.BlockSpec((NC, R, D), lambda i: (0, i, 0)),
            pl.BlockSpec((R, D), lambda i: (i, 0)),
            pl.BlockSpec((D, D), lambda i: (0, 0)),
        ],
        out_specs=[
            pl.BlockSpec((R, D), lambda i: (i, 0)),
            pl.BlockSpec((R, D), lambda i: (i, 0)),
        ],
        out_shape=[
            jax.ShapeDtypeStruct((NP, D), jnp.float32),
            jax.ShapeDtypeStruct((NP, D), jnp.float32),
        ],
    )(degs, x_p, w_enc)


def _tc_stage(s, y, dinv, b, h_prev, w_next, *, smooth, use_relu, width,
              width_next):
    have_h = h_prev is not None
    have_w = w_next is not None

    def body(*refs):
        i = 0
        s_ref = refs[i]; i += 1
        y_ref = refs[i]; i += 1
        dinv_ref = refs[i]; i += 1
        b_ref = refs[i]; i += 1
        h_ref = refs[i] if have_h else None
        i += have_h
        w_ref = refs[i] if have_w else None
        i += have_w
        out_refs = refs[i:]
        dv = dinv_ref[...]
        z = s_ref[0] + s_ref[1] - y_ref[...]
        c = z * dv[:, :width] + b_ref[...]
        a = jnp.maximum(c, 0.0) if use_relu else c
        h = smooth * h_ref[...] + (1.0 - smooth) * a if have_h else a
        out_refs[0][...] = h
        if have_w:
            u = jnp.dot(h, w_ref[...], preferred_element_type=jnp.float32)
            out_refs[1][...] = u * dv[:, :width_next]

    in_specs = [
        pl.BlockSpec((NC, R, width), lambda i: (0, i, 0)),
        pl.BlockSpec((R, width), lambda i: (i, 0)),
        pl.BlockSpec((R, D), lambda i: (i, 0)),
        pl.BlockSpec((1, width), lambda i: (0, 0)),
    ]
    args = [s, y, dinv, b]
    if have_h:
        in_specs.append(pl.BlockSpec((R, width), lambda i: (i, 0)))
        args.append(h_prev)
    if have_w:
        in_specs.append(pl.BlockSpec((width, width_next), lambda i: (0, 0)))
        args.append(w_next)
    out_specs = [pl.BlockSpec((R, width), lambda i: (i, 0))]
    out_shape = [jax.ShapeDtypeStruct((NP, width), jnp.float32)]
    if have_w:
        out_specs.append(pl.BlockSpec((R, width_next), lambda i: (i, 0)))
        out_shape.append(jax.ShapeDtypeStruct((NP, width_next), jnp.float32))
    res = pl.pallas_call(
        body, grid=(NB,), in_specs=in_specs, out_specs=out_specs,
        out_shape=out_shape,
    )(*args)
    return res if have_w else (res[0], None)


def kernel(x, edge_index, W_enc, b_enc, W_gc, b_gc, W_dec, b_dec):
    schedule = (0.5, 0.5, 0.5, 0.5)
    src = edge_index[0].astype(jnp.int32)
    dst = edge_index[1].astype(jnp.int32)
    pad = N + jnp.arange(EP - E, dtype=jnp.int32) % (NP - N)
    srcc = jnp.concatenate([src, pad]).reshape(NW, NCH, K)
    dstc = jnp.concatenate([dst, pad]).reshape(NW, NCH, K)
    x_p = jnp.pad(x, ((0, NP - N), (0, 0)))
    w_dec_p = jnp.pad(W_dec, ((0, 0), (0, DD - D_OUT)))
    b_dec_p = jnp.pad(b_dec, (0, DD - D_OUT)).reshape(1, DD)
    b_enc2 = b_enc.reshape(1, D)
    b_gc2 = b_gc.reshape(1, D)
    ones_np = jnp.ones((NP, D), jnp.float32)

    spmm = _make_spmm(D)
    spmm_dec = spmm

    degs = _make_deg()(dstc, ones_np)
    dinv, y = _tc_pre(degs, x_p, W_enc)

    s = spmm(srcc, dstc, y)
    h, y = _tc_stage(s, y, dinv, b_enc2, None, W_gc,
                     smooth=0.0, use_relu=True, width=D, width_next=D)
    for it, sf in enumerate(schedule):
        last = it == len(schedule) - 1
        w_next = w_dec_p if last else W_gc
        wn = DD if last else D
        s = spmm(srcc, dstc, y)
        h, y = _tc_stage(s, y, dinv, b_gc2, h, w_next,
                         smooth=sf, use_relu=True, width=D, width_next=wn)
    s = spmm_dec(srcc, dstc, y)
    out, _ = _tc_stage(s, y, dinv, b_dec_p, None, None,
                       smooth=0.0, use_relu=False, width=DD, width_next=DD)
    return out[:N, :D_OUT]
